# baseline (speedup 1.0000x reference)
import functools

import jax
import jax.numpy as jnp
from jax import lax
from jax.experimental import pallas as pl
from jax.experimental.pallas import tpu as pltpu
from jax.experimental.pallas import tpu_sc as plsc

N = 10000
E = 320000
F = 128

NC, NS = 2, 16
NW = NC * NS
EPW = E // NW
CH = 80
NBUF = 4
NCH = 128
EPAD = NCH * CH
PAD = EPAD - EPW
RPW = 626
NACC = NS * RPW
MB = 1000
VL = 16

assert NCH % NBUF == 0 and EPAD >= EPW


def _mm_body(x_ref, w_ref, o_ref):
    o_ref[...] = jnp.dot(x_ref[...], w_ref[...],
                         preferred_element_type=jnp.float32)


_matmul = pl.pallas_call(
    _mm_body,
    grid=(N // MB,),
    in_specs=[
        pl.BlockSpec((MB, F), lambda i: (i, 0)),
        pl.BlockSpec((F, F), lambda i: (0, 0)),
    ],
    out_specs=pl.BlockSpec((MB, F), lambda i: (i, 0)),
    out_shape=jax.ShapeDtypeStruct((N, F), jnp.float32),
)


def _comb_body(p_ref, b_ref, o_ref):
    o_ref[...] = p_ref[0] + p_ref[1] + b_ref[...]


_combine = pl.pallas_call(
    _comb_body,
    grid=(N // MB,),
    in_specs=[
        pl.BlockSpec((NC, MB, F), lambda i: (0, i, 0)),
        pl.BlockSpec((1, F), lambda i: (0, 0)),
    ],
    out_specs=pl.BlockSpec((MB, F), lambda i: (i, 0)),
    out_shape=jax.ShapeDtypeStruct((N, F), jnp.float32),
)


@functools.partial(
    pl.kernel,
    out_type=jax.ShapeDtypeStruct((NW, RPW, F), jnp.float32),
    mesh=plsc.VectorSubcoreMesh(core_axis_name="c", subcore_axis_name="s"),
    scratch_types=[
        [pltpu.VMEM((CH,), jnp.int32) for _ in range(NBUF)],
        [pltpu.VMEM((CH,), jnp.int32) for _ in range(NBUF)],
        [pltpu.VMEM((CH,), jnp.int32) for _ in range(NBUF)],
        [pltpu.VMEM((CH, F), jnp.float32) for _ in range(NBUF)],
        pltpu.VMEM_SHARED((NACC, F), jnp.float32),
        [pltpu.SemaphoreType.DMA for _ in range(NBUF)],
        [pltpu.SemaphoreType.DMA for _ in range(NBUF)],
    ],
)
def _sc_aggregate(edge_hbm, sup_hbm, out_hbm,
                  pcks, srcs, dsts, rows, acc, sems, psems):
    cid = lax.axis_index("c")
    sid = lax.axis_index("s")
    w = cid * NS + sid

    def zrow(r, carry):
        for c in range(F // VL):
            rows[0][r, pl.ds(c * VL, VL)] = jnp.zeros((VL,), jnp.float32)
        return carry

    lax.fori_loop(0, CH, zrow, 0)
    for j in range(RPW // CH):
        pltpu.sync_copy(rows[0], acc.at[pl.ds(sid * RPW + j * CH, CH)])
    _tail = RPW % CH
    if _tail:
        pltpu.sync_copy(rows[0].at[pl.ds(0, _tail)],
                        acc.at[pl.ds(sid * RPW + RPW - _tail, _tail)])
    plsc.subcore_barrier()

    def unpack(pref, sref, dref):
        for c in range(CH // VL):
            v = pref[pl.ds(c * VL, VL)]
            sref[pl.ds(c * VL, VL)] = lax.bitwise_and(v, 0xFFFF)
            dref[pl.ds(c * VL, VL)] = lax.shift_right_logical(v, 16)

    for j in range(NBUF):
        pltpu.async_copy(edge_hbm.at[w, j], pcks[j], psems[j])
    for j in range(NBUF):
        pltpu.make_async_copy(edge_hbm.at[w, j], pcks[j], psems[j]).wait()
        unpack(pcks[j], srcs[j], dsts[j])
        pltpu.async_copy(sup_hbm.at[srcs[j]], rows[j], sems[j])
        pltpu.async_copy(edge_hbm.at[w, j + NBUF], pcks[j], psems[j])

    T = NCH // NBUF

    def body(t, carry):
        for j in range(NBUF):
            k = NBUF * t + j
            pltpu.make_async_copy(sup_hbm.at[srcs[j]], rows[j], sems[j]).wait()
            pltpu.sync_copy(rows[j], acc.at[dsts[j]], add=True)

            @pl.when(t < T - 1)
            def _next_gather():
                pltpu.make_async_copy(
                    edge_hbm.at[w, k + NBUF], pcks[j], psems[j]).wait()
                unpack(pcks[j], srcs[j], dsts[j])
                pltpu.async_copy(sup_hbm.at[srcs[j]], rows[j], sems[j])

            @pl.when(t < T - 2)
            def _next_pck():
                pltpu.async_copy(
                    edge_hbm.at[w, k + 2 * NBUF], pcks[j], psems[j])

        return carry

    lax.fori_loop(0, T, body, 0)
    plsc.subcore_barrier()

    pltpu.sync_copy(acc.at[pl.ds(sid * RPW, RPW)], out_hbm.at[w])


def kernel(input, edge_index, weight, bias):
    x = input.astype(jnp.float32)
    wt = weight.astype(jnp.float32)
    src = edge_index[0].astype(jnp.int32).reshape(NW, EPW)
    dst = edge_index[1].astype(jnp.int32).reshape(NW, EPW)
    pad_src = jnp.zeros((NW, PAD), jnp.int32)
    pad_dst = jnp.broadcast_to(
        N + (jnp.arange(PAD, dtype=jnp.int32) % (NACC - N)), (NW, PAD))
    src_p = jnp.concatenate([src, pad_src], axis=1)
    dst_p = jnp.concatenate([dst, pad_dst], axis=1)
    packed = (src_p | (dst_p << 16)).reshape(NW, NCH, CH)

    support = _matmul(x, wt)
    parts = _sc_aggregate(packed, support)
    parts = parts.reshape(NC, NS * RPW, F)[:, :N]
    return _combine(parts, bias.reshape(1, F).astype(jnp.float32))
